# Initial kernel scaffold; baseline (speedup 1.0000x reference)
import jax
import jax.numpy as jnp
from jax.experimental import pallas as pl


def kernel(points1, points2, W0, b0, W1, b1, W2, b2):
    raise NotImplementedError("write your pallas kernel here")


# speedup vs baseline: 22.8652x; 22.8652x over previous
import functools

import jax
import jax.numpy as jnp
from jax.experimental import pallas as pl
from jax.experimental.pallas import tpu as pltpu

K_NN = 3
Q_BLK = 512


def _fused_kernel(p1_ref, p2_ref, w0_ref, b0_ref, w1_ref, b1_ref,
                  w2_ref, b2_ref, out_ref):
    p1 = p1_ref[0]
    p2 = p2_ref[0]
    xyz1 = p1[:, :3]
    feat1 = p1[:, 3:]
    xyz2 = p2[:, :3]
    feat2 = p2[:, 3:]

    q = xyz1.shape[0]
    n2 = xyz2.shape[0]

    a2 = jnp.sum(xyz1 * xyz1, axis=1, keepdims=True)
    b2v = jnp.sum(xyz2 * xyz2, axis=1, keepdims=True).T
    cross = jax.lax.dot_general(
        xyz1, xyz2, (((1,), (1,)), ((), ())),
        preferred_element_type=jnp.float32)
    d2 = a2 + b2v - 2.0 * cross
    dist = jnp.sqrt(jnp.maximum(d2, 1e-12))

    col = jax.lax.broadcasted_iota(jnp.int32, (q, n2), 1)

    work = dist
    sel = []
    for _ in range(K_NN):
        m = jnp.min(work, axis=1, keepdims=True)
        tied = work <= m
        idx = jnp.min(jnp.where(tied, col, n2), axis=1, keepdims=True)
        oh = col == idx
        sel.append((oh, m))
        work = jnp.where(oh, jnp.float32(jnp.inf), work)

    ws = [1.0 / (m + 1e-8) for _, m in sel]
    wsum = ws[0] + ws[1] + ws[2]
    s = jnp.zeros((q, n2), jnp.float32)
    for (oh, _), w in zip(sel, ws):
        s = s + jnp.where(oh, w / wsum, 0.0)

    interp = jax.lax.dot_general(
        s, feat2, (((1,), (0,)), ((), ())),
        preferred_element_type=jnp.float32)

    h = jnp.concatenate([feat1, interp], axis=1)
    h = jnp.maximum(jnp.dot(h, w0_ref[:], preferred_element_type=jnp.float32)
                    + b0_ref[:], 0.0)
    h = jnp.maximum(jnp.dot(h, w1_ref[:], preferred_element_type=jnp.float32)
                    + b1_ref[:], 0.0)
    h = jnp.maximum(jnp.dot(h, w2_ref[:], preferred_element_type=jnp.float32)
                    + b2_ref[:], 0.0)
    out_ref[0] = jnp.concatenate([xyz1, h], axis=1)


@jax.jit
def kernel(points1, points2, W0, b0, W1, b1, W2, b2):
    B, N1, C1 = points1.shape
    _, N2, C2 = points2.shape
    grid = (B, N1 // Q_BLK)

    out = pl.pallas_call(
        _fused_kernel,
        grid=grid,
        in_specs=[
            pl.BlockSpec((1, Q_BLK, C1), lambda b, i: (b, i, 0)),
            pl.BlockSpec((1, N2, C2), lambda b, i: (b, 0, 0)),
            pl.BlockSpec(W0.shape, lambda b, i: (0, 0)),
            pl.BlockSpec((1, b0.shape[0]), lambda b, i: (0, 0)),
            pl.BlockSpec(W1.shape, lambda b, i: (0, 0)),
            pl.BlockSpec((1, b1.shape[0]), lambda b, i: (0, 0)),
            pl.BlockSpec(W2.shape, lambda b, i: (0, 0)),
            pl.BlockSpec((1, b2.shape[0]), lambda b, i: (0, 0)),
        ],
        out_specs=pl.BlockSpec((1, Q_BLK, 3 + W2.shape[1]),
                               lambda b, i: (b, i, 0)),
        out_shape=jax.ShapeDtypeStruct((B, N1, 3 + W2.shape[1]),
                                       jnp.float32),
        compiler_params=pltpu.CompilerParams(
            dimension_semantics=("parallel", "parallel")),
    )(points1, points2, W0, b0.reshape(1, -1), W1, b1.reshape(1, -1),
      W2, b2.reshape(1, -1))
    return out


# speedup vs baseline: 24.4181x; 1.0679x over previous
import functools

import jax
import jax.numpy as jnp
from jax.experimental import pallas as pl
from jax.experimental.pallas import tpu as pltpu

K_NN = 3
Q_BLK = 512


def _fused_kernel(p1_ref, p2_ref, w0_ref, b0_ref, w1_ref, b1_ref,
                  w2_ref, b2_ref, out_ref):
    p1 = p1_ref[0]
    p2 = p2_ref[0]
    xyz1 = p1[:, :3]
    feat1 = p1[:, 3:]
    xyz2 = p2[:, :3]
    feat2 = p2[:, 3:]

    q = xyz1.shape[0]
    n2 = xyz2.shape[0]

    a2 = jnp.sum(xyz1 * xyz1, axis=1, keepdims=True)
    b2v = jnp.sum(xyz2 * xyz2, axis=1, keepdims=True).T
    cross = jax.lax.dot_general(
        xyz1, xyz2, (((1,), (1,)), ((), ())),
        preferred_element_type=jnp.float32)
    d2 = a2 + b2v - 2.0 * cross

    col = jax.lax.broadcasted_iota(jnp.int32, (q, n2), 1)

    work = d2
    sel = []
    for _ in range(K_NN):
        m = jnp.min(work, axis=1, keepdims=True)
        idx = jnp.min(jnp.where(work <= m, col, n2), axis=1, keepdims=True)
        oh = col == idx
        sel.append((oh, m))
        work = jnp.where(oh, jnp.float32(jnp.inf), work)

    ws = [1.0 / (jnp.sqrt(jnp.maximum(m, 1e-12)) + 1e-8) for _, m in sel]
    wsum = ws[0] + ws[1] + ws[2]
    s = jnp.zeros((q, n2), jnp.float32)
    for (oh, _), w in zip(sel, ws):
        s = s + jnp.where(oh, w / wsum, 0.0)

    interp = jax.lax.dot_general(
        s, feat2, (((1,), (0,)), ((), ())),
        preferred_element_type=jnp.float32)

    h = jnp.maximum(
        jax.lax.dot_general(feat1, w0_ref[:13, :], (((1,), (0,)), ((), ())),
                            preferred_element_type=jnp.float32)
        + jax.lax.dot_general(interp, w0_ref[13:, :], (((1,), (0,)), ((), ())),
                              preferred_element_type=jnp.float32)
        + b0_ref[:], 0.0)
    h = jnp.maximum(jnp.dot(h, w1_ref[:], preferred_element_type=jnp.float32)
                    + b1_ref[:], 0.0)
    h = jnp.maximum(jnp.dot(h, w2_ref[:], preferred_element_type=jnp.float32)
                    + b2_ref[:], 0.0)
    out_ref[0] = jnp.concatenate([xyz1, h], axis=1)


@jax.jit
def kernel(points1, points2, W0, b0, W1, b1, W2, b2):
    B, N1, C1 = points1.shape
    _, N2, C2 = points2.shape
    grid = (B, N1 // Q_BLK)

    out = pl.pallas_call(
        _fused_kernel,
        grid=grid,
        in_specs=[
            pl.BlockSpec((1, Q_BLK, C1), lambda b, i: (b, i, 0)),
            pl.BlockSpec((1, N2, C2), lambda b, i: (b, 0, 0)),
            pl.BlockSpec(W0.shape, lambda b, i: (0, 0)),
            pl.BlockSpec((1, b0.shape[0]), lambda b, i: (0, 0)),
            pl.BlockSpec(W1.shape, lambda b, i: (0, 0)),
            pl.BlockSpec((1, b1.shape[0]), lambda b, i: (0, 0)),
            pl.BlockSpec(W2.shape, lambda b, i: (0, 0)),
            pl.BlockSpec((1, b2.shape[0]), lambda b, i: (0, 0)),
        ],
        out_specs=pl.BlockSpec((1, Q_BLK, 3 + W2.shape[1]),
                               lambda b, i: (b, i, 0)),
        out_shape=jax.ShapeDtypeStruct((B, N1, 3 + W2.shape[1]),
                                       jnp.float32),
        compiler_params=pltpu.CompilerParams(
            dimension_semantics=("parallel", "parallel")),
    )(points1, points2, W0, b0.reshape(1, -1), W1, b1.reshape(1, -1),
      W2, b2.reshape(1, -1))
    return out


# speedup vs baseline: 30.3606x; 1.2434x over previous
import functools

import jax
import jax.numpy as jnp
from jax.experimental import pallas as pl
from jax.experimental.pallas import tpu as pltpu

K_NN = 3
Q_BLK = 512


def _fused_kernel(p1_ref, p2_ref, w0_ref, b0_ref, w1_ref, b1_ref,
                  w2_ref, b2_ref, out_ref):
    p1 = p1_ref[0]
    p2 = p2_ref[0]
    xyz1 = p1[:, :3]
    feat1 = p1[:, 3:]
    xyz2 = p2[:, :3]
    feat2 = p2[:, 3:]

    q = xyz1.shape[0]
    n2 = xyz2.shape[0]

    a2 = jnp.sum(xyz1 * xyz1, axis=1, keepdims=True)
    b2v = jnp.sum(xyz2 * xyz2, axis=1, keepdims=True).T
    cross = jax.lax.dot_general(
        xyz1, xyz2, (((1,), (1,)), ((), ())),
        preferred_element_type=jnp.float32)
    d2 = a2 + b2v - 2.0 * cross

    inf = jnp.float32(jnp.inf)
    m1 = jnp.min(d2, axis=1, keepdims=True)
    m2 = jnp.min(jnp.where(d2 <= m1, inf, d2), axis=1, keepdims=True)
    m3 = jnp.min(jnp.where(d2 <= m2, inf, d2), axis=1, keepdims=True)

    def _w(v):
        return 1.0 / (jnp.sqrt(jnp.maximum(v, 1e-12)) + 1e-8)
    inv_wsum = 1.0 / (_w(m1) + _w(m2) + _w(m3))
    s = jnp.where(d2 <= m3, _w(d2) * inv_wsum, 0.0)

    interp = jax.lax.dot_general(
        s, feat2, (((1,), (0,)), ((), ())),
        preferred_element_type=jnp.float32)

    h = jnp.maximum(
        jax.lax.dot_general(feat1, w0_ref[:13, :], (((1,), (0,)), ((), ())),
                            preferred_element_type=jnp.float32)
        + jax.lax.dot_general(interp, w0_ref[13:, :], (((1,), (0,)), ((), ())),
                              preferred_element_type=jnp.float32)
        + b0_ref[:], 0.0)
    h = jnp.maximum(jnp.dot(h, w1_ref[:], preferred_element_type=jnp.float32)
                    + b1_ref[:], 0.0)
    h = jnp.maximum(jnp.dot(h, w2_ref[:], preferred_element_type=jnp.float32)
                    + b2_ref[:], 0.0)
    out_ref[0] = jnp.concatenate([xyz1, h], axis=1)


@jax.jit
def kernel(points1, points2, W0, b0, W1, b1, W2, b2):
    B, N1, C1 = points1.shape
    _, N2, C2 = points2.shape
    grid = (B, N1 // Q_BLK)

    out = pl.pallas_call(
        _fused_kernel,
        grid=grid,
        in_specs=[
            pl.BlockSpec((1, Q_BLK, C1), lambda b, i: (b, i, 0)),
            pl.BlockSpec((1, N2, C2), lambda b, i: (b, 0, 0)),
            pl.BlockSpec(W0.shape, lambda b, i: (0, 0)),
            pl.BlockSpec((1, b0.shape[0]), lambda b, i: (0, 0)),
            pl.BlockSpec(W1.shape, lambda b, i: (0, 0)),
            pl.BlockSpec((1, b1.shape[0]), lambda b, i: (0, 0)),
            pl.BlockSpec(W2.shape, lambda b, i: (0, 0)),
            pl.BlockSpec((1, b2.shape[0]), lambda b, i: (0, 0)),
        ],
        out_specs=pl.BlockSpec((1, Q_BLK, 3 + W2.shape[1]),
                               lambda b, i: (b, i, 0)),
        out_shape=jax.ShapeDtypeStruct((B, N1, 3 + W2.shape[1]),
                                       jnp.float32),
        compiler_params=pltpu.CompilerParams(
            dimension_semantics=("parallel", "parallel")),
    )(points1, points2, W0, b0.reshape(1, -1), W1, b1.reshape(1, -1),
      W2, b2.reshape(1, -1))
    return out


# speedup vs baseline: 38.1067x; 1.2551x over previous
import functools

import jax
import jax.numpy as jnp
from jax.experimental import pallas as pl
from jax.experimental.pallas import tpu as pltpu

K_NN = 3
Q_BLK = 512


def _fused_kernel(p1_ref, p2_ref, w0_ref, b0_ref, w1_ref, b1_ref,
                  w2_ref, b2_ref, out_ref):
    p1 = p1_ref[0]
    p2 = p2_ref[0]
    xyz1 = p1[:, :3]
    feat1 = p1[:, 3:]
    xyz2 = p2[:, :3]
    feat2 = p2[:, 3:]

    q = xyz1.shape[0]
    n2 = xyz2.shape[0]

    a2 = jnp.sum(xyz1 * xyz1, axis=1, keepdims=True)
    b2c = jnp.sum(xyz2 * xyz2, axis=1, keepdims=True)
    ones_q = jnp.ones((q, 1), jnp.float32)
    ones_n = jnp.ones((n2, 1), jnp.float32)
    aug1 = jnp.concatenate([-2.0 * xyz1, ones_q, a2], axis=1)
    aug2 = jnp.concatenate([xyz2, b2c, ones_n], axis=1)
    d2 = jax.lax.dot_general(
        aug1, aug2, (((1,), (1,)), ((), ())),
        preferred_element_type=jnp.float32)

    inf = jnp.float32(jnp.inf)
    m1 = jnp.min(d2, axis=1, keepdims=True)
    m2 = jnp.min(jnp.where(d2 <= m1, inf, d2), axis=1, keepdims=True)
    m3 = jnp.min(jnp.where(d2 <= m2, inf, d2), axis=1, keepdims=True)

    def _w(v):
        return jax.lax.rsqrt(jnp.maximum(v, 1e-12))
    inv_wsum = 1.0 / (_w(m1) + _w(m2) + _w(m3))
    s = jnp.where(d2 <= m3, _w(d2) * inv_wsum, 0.0)

    interp = jax.lax.dot_general(
        s, feat2, (((1,), (0,)), ((), ())),
        preferred_element_type=jnp.float32)

    h = jnp.maximum(
        jax.lax.dot_general(feat1, w0_ref[:13, :], (((1,), (0,)), ((), ())),
                            preferred_element_type=jnp.float32)
        + jax.lax.dot_general(interp, w0_ref[13:, :], (((1,), (0,)), ((), ())),
                              preferred_element_type=jnp.float32)
        + b0_ref[:], 0.0)
    h = jnp.maximum(jnp.dot(h, w1_ref[:], preferred_element_type=jnp.float32)
                    + b1_ref[:], 0.0)
    h = jnp.maximum(jnp.dot(h, w2_ref[:], preferred_element_type=jnp.float32)
                    + b2_ref[:], 0.0)
    out_ref[0] = jnp.concatenate([xyz1, h], axis=1)


@jax.jit
def kernel(points1, points2, W0, b0, W1, b1, W2, b2):
    B, N1, C1 = points1.shape
    _, N2, C2 = points2.shape
    grid = (B, N1 // Q_BLK)

    out = pl.pallas_call(
        _fused_kernel,
        grid=grid,
        in_specs=[
            pl.BlockSpec((1, Q_BLK, C1), lambda b, i: (b, i, 0)),
            pl.BlockSpec((1, N2, C2), lambda b, i: (b, 0, 0)),
            pl.BlockSpec(W0.shape, lambda b, i: (0, 0)),
            pl.BlockSpec((1, b0.shape[0]), lambda b, i: (0, 0)),
            pl.BlockSpec(W1.shape, lambda b, i: (0, 0)),
            pl.BlockSpec((1, b1.shape[0]), lambda b, i: (0, 0)),
            pl.BlockSpec(W2.shape, lambda b, i: (0, 0)),
            pl.BlockSpec((1, b2.shape[0]), lambda b, i: (0, 0)),
        ],
        out_specs=pl.BlockSpec((1, Q_BLK, 3 + W2.shape[1]),
                               lambda b, i: (b, i, 0)),
        out_shape=jax.ShapeDtypeStruct((B, N1, 3 + W2.shape[1]),
                                       jnp.float32),
        compiler_params=pltpu.CompilerParams(
            dimension_semantics=("parallel", "parallel")),
    )(points1, points2, W0, b0.reshape(1, -1), W1, b1.reshape(1, -1),
      W2, b2.reshape(1, -1))
    return out


# speedup vs baseline: 38.9562x; 1.0223x over previous
import functools

import jax
import jax.numpy as jnp
from jax.experimental import pallas as pl
from jax.experimental.pallas import tpu as pltpu

K_NN = 3
Q_BLK = 512


def _fused_kernel(p1_ref, p2_ref, w0_ref, b0_ref, w1_ref, b1_ref,
                  w2_ref, b2_ref, out_ref):
    p1 = p1_ref[0]
    p2t = p2_ref[0]
    xyz1 = p1[:, :3]
    feat1 = p1[:, 3:]
    xyz2t = p2t[:3, :]
    feat2t = p2t[3:, :]

    q = xyz1.shape[0]
    n2 = xyz2t.shape[1]

    a2 = jnp.sum(xyz1 * xyz1, axis=1, keepdims=True)
    b2row = jnp.sum(xyz2t * xyz2t, axis=0, keepdims=True)
    cross = jax.lax.dot_general(
        xyz1, xyz2t, (((1,), (0,)), ((), ())),
        preferred_element_type=jnp.float32)
    d2 = a2 + b2row - 2.0 * cross

    inf = jnp.float32(jnp.inf)
    m1 = jnp.min(d2, axis=1, keepdims=True)
    m2 = jnp.min(jnp.where(d2 <= m1, inf, d2), axis=1, keepdims=True)
    m3 = jnp.min(jnp.where(d2 <= m2, inf, d2), axis=1, keepdims=True)

    def _w(v):
        return jax.lax.rsqrt(jnp.maximum(v, 1e-12))
    inv_wsum = 1.0 / (_w(m1) + _w(m2) + _w(m3))
    s = jnp.where(d2 <= m3, _w(d2) * inv_wsum, 0.0)

    interp = jax.lax.dot_general(
        s, feat2t, (((1,), (1,)), ((), ())),
        preferred_element_type=jnp.float32)

    h = jnp.maximum(
        jax.lax.dot_general(feat1, w0_ref[:13, :], (((1,), (0,)), ((), ())),
                            preferred_element_type=jnp.float32)
        + jax.lax.dot_general(interp, w0_ref[13:, :], (((1,), (0,)), ((), ())),
                              preferred_element_type=jnp.float32)
        + b0_ref[:], 0.0)
    h = jnp.maximum(jnp.dot(h, w1_ref[:], preferred_element_type=jnp.float32)
                    + b1_ref[:], 0.0)
    h = jnp.maximum(jnp.dot(h, w2_ref[:], preferred_element_type=jnp.float32)
                    + b2_ref[:], 0.0)
    out_ref[0] = jnp.concatenate([xyz1, h], axis=1)


@jax.jit
def kernel(points1, points2, W0, b0, W1, b1, W2, b2):
    B, N1, C1 = points1.shape
    _, N2, C2 = points2.shape
    grid = (B, N1 // Q_BLK)

    out = pl.pallas_call(
        _fused_kernel,
        grid=grid,
        in_specs=[
            pl.BlockSpec((1, Q_BLK, C1), lambda b, i: (b, i, 0)),
            pl.BlockSpec((1, C2, N2), lambda b, i: (b, 0, 0)),
            pl.BlockSpec(W0.shape, lambda b, i: (0, 0)),
            pl.BlockSpec((1, b0.shape[0]), lambda b, i: (0, 0)),
            pl.BlockSpec(W1.shape, lambda b, i: (0, 0)),
            pl.BlockSpec((1, b1.shape[0]), lambda b, i: (0, 0)),
            pl.BlockSpec(W2.shape, lambda b, i: (0, 0)),
            pl.BlockSpec((1, b2.shape[0]), lambda b, i: (0, 0)),
        ],
        out_specs=pl.BlockSpec((1, Q_BLK, 3 + W2.shape[1]),
                               lambda b, i: (b, i, 0)),
        out_shape=jax.ShapeDtypeStruct((B, N1, 3 + W2.shape[1]),
                                       jnp.float32),
        compiler_params=pltpu.CompilerParams(
            dimension_semantics=("parallel", "parallel")),
    )(points1, jnp.swapaxes(points2, 1, 2), W0, b0.reshape(1, -1),
      W1, b1.reshape(1, -1), W2, b2.reshape(1, -1))
    return out


# speedup vs baseline: 45.2616x; 1.1619x over previous
import functools

import jax
import jax.numpy as jnp
from jax.experimental import pallas as pl
from jax.experimental.pallas import tpu as pltpu

K_NN = 3
Q_BLK = 1024


def _fused_kernel(p1_ref, p2_ref, w0_ref, b0_ref, w1_ref, b1_ref,
                  w2_ref, b2_ref, out_ref):
    p1 = p1_ref[0]
    p2t = p2_ref[0]
    xyz1 = p1[:, :3]
    feat1 = p1[:, 3:]
    xyz2t = p2t[:3, :]
    feat2t = p2t[3:, :]

    q = xyz1.shape[0]
    n2 = xyz2t.shape[1]

    a2 = jnp.sum(xyz1 * xyz1, axis=1, keepdims=True)
    b2row = jnp.sum(xyz2t * xyz2t, axis=0, keepdims=True)
    cross = jax.lax.dot_general(
        xyz1, xyz2t, (((1,), (0,)), ((), ())),
        preferred_element_type=jnp.float32)
    d2 = a2 + b2row - 2.0 * cross

    inf = jnp.float32(jnp.inf)
    m1 = jnp.min(d2, axis=1, keepdims=True)
    m2 = jnp.min(jnp.where(d2 <= m1, inf, d2), axis=1, keepdims=True)
    m3 = jnp.min(jnp.where(d2 <= m2, inf, d2), axis=1, keepdims=True)

    def _w(v):
        return jax.lax.rsqrt(jnp.maximum(v, 1e-12))
    inv_wsum = 1.0 / (_w(m1) + _w(m2) + _w(m3))
    s = jnp.where(d2 <= m3, _w(d2) * inv_wsum, 0.0)

    interp = jax.lax.dot_general(
        s, feat2t, (((1,), (1,)), ((), ())),
        preferred_element_type=jnp.float32)

    h = jnp.maximum(
        jax.lax.dot_general(feat1, w0_ref[:13, :], (((1,), (0,)), ((), ())),
                            preferred_element_type=jnp.float32)
        + jax.lax.dot_general(interp, w0_ref[13:, :], (((1,), (0,)), ((), ())),
                              preferred_element_type=jnp.float32)
        + b0_ref[:], 0.0)
    h = jnp.maximum(jnp.dot(h, w1_ref[:], preferred_element_type=jnp.float32)
                    + b1_ref[:], 0.0)
    h = jnp.maximum(jnp.dot(h, w2_ref[:], preferred_element_type=jnp.float32)
                    + b2_ref[:], 0.0)
    out_ref[0] = jnp.concatenate([xyz1, h], axis=1)


@jax.jit
def kernel(points1, points2, W0, b0, W1, b1, W2, b2):
    B, N1, C1 = points1.shape
    _, N2, C2 = points2.shape
    grid = (B, N1 // Q_BLK)

    out = pl.pallas_call(
        _fused_kernel,
        grid=grid,
        in_specs=[
            pl.BlockSpec((1, Q_BLK, C1), lambda b, i: (b, i, 0)),
            pl.BlockSpec((1, C2, N2), lambda b, i: (b, 0, 0)),
            pl.BlockSpec(W0.shape, lambda b, i: (0, 0)),
            pl.BlockSpec((1, b0.shape[0]), lambda b, i: (0, 0)),
            pl.BlockSpec(W1.shape, lambda b, i: (0, 0)),
            pl.BlockSpec((1, b1.shape[0]), lambda b, i: (0, 0)),
            pl.BlockSpec(W2.shape, lambda b, i: (0, 0)),
            pl.BlockSpec((1, b2.shape[0]), lambda b, i: (0, 0)),
        ],
        out_specs=pl.BlockSpec((1, Q_BLK, 3 + W2.shape[1]),
                               lambda b, i: (b, i, 0)),
        out_shape=jax.ShapeDtypeStruct((B, N1, 3 + W2.shape[1]),
                                       jnp.float32),
        compiler_params=pltpu.CompilerParams(
            dimension_semantics=("parallel", "parallel")),
    )(points1, jnp.swapaxes(points2, 1, 2), W0, b0.reshape(1, -1),
      W1, b1.reshape(1, -1), W2, b2.reshape(1, -1))
    return out


# speedup vs baseline: 48.2193x; 1.0653x over previous
import functools

import jax
import jax.numpy as jnp
from jax.experimental import pallas as pl
from jax.experimental.pallas import tpu as pltpu

K_NN = 3
Q_BLK = 2048


def _fused_kernel(p1_ref, p2_ref, w0_ref, b0_ref, w1_ref, b1_ref,
                  w2_ref, b2_ref, out_ref):
    p1 = p1_ref[0]
    p2t = p2_ref[0]
    xyz1 = p1[:, :3]
    feat1 = p1[:, 3:]
    xyz2t = p2t[:3, :]
    feat2t = p2t[3:, :]

    q = xyz1.shape[0]
    n2 = xyz2t.shape[1]

    a2 = jnp.sum(xyz1 * xyz1, axis=1, keepdims=True)
    b2row = jnp.sum(xyz2t * xyz2t, axis=0, keepdims=True)
    cross = jax.lax.dot_general(
        xyz1, xyz2t, (((1,), (0,)), ((), ())),
        preferred_element_type=jnp.float32)
    d2 = a2 + b2row - 2.0 * cross

    inf = jnp.float32(jnp.inf)
    m1 = jnp.min(d2, axis=1, keepdims=True)
    m2 = jnp.min(jnp.where(d2 <= m1, inf, d2), axis=1, keepdims=True)
    m3 = jnp.min(jnp.where(d2 <= m2, inf, d2), axis=1, keepdims=True)

    def _w(v):
        return jax.lax.rsqrt(jnp.maximum(v, 1e-12))
    inv_wsum = 1.0 / (_w(m1) + _w(m2) + _w(m3))
    s = jnp.where(d2 <= m3, _w(d2) * inv_wsum, 0.0)

    interp = jax.lax.dot_general(
        s, feat2t, (((1,), (1,)), ((), ())),
        preferred_element_type=jnp.float32)

    h = jnp.maximum(
        jax.lax.dot_general(feat1, w0_ref[:13, :], (((1,), (0,)), ((), ())),
                            preferred_element_type=jnp.float32)
        + jax.lax.dot_general(interp, w0_ref[13:, :], (((1,), (0,)), ((), ())),
                              preferred_element_type=jnp.float32)
        + b0_ref[:], 0.0)
    h = jnp.maximum(jnp.dot(h, w1_ref[:], preferred_element_type=jnp.float32)
                    + b1_ref[:], 0.0)
    h = jnp.maximum(jnp.dot(h, w2_ref[:], preferred_element_type=jnp.float32)
                    + b2_ref[:], 0.0)
    out_ref[0] = jnp.concatenate([xyz1, h], axis=1)


@jax.jit
def kernel(points1, points2, W0, b0, W1, b1, W2, b2):
    B, N1, C1 = points1.shape
    _, N2, C2 = points2.shape
    grid = (B, N1 // Q_BLK)

    out = pl.pallas_call(
        _fused_kernel,
        grid=grid,
        in_specs=[
            pl.BlockSpec((1, Q_BLK, C1), lambda b, i: (b, i, 0)),
            pl.BlockSpec((1, C2, N2), lambda b, i: (b, 0, 0)),
            pl.BlockSpec(W0.shape, lambda b, i: (0, 0)),
            pl.BlockSpec((1, b0.shape[0]), lambda b, i: (0, 0)),
            pl.BlockSpec(W1.shape, lambda b, i: (0, 0)),
            pl.BlockSpec((1, b1.shape[0]), lambda b, i: (0, 0)),
            pl.BlockSpec(W2.shape, lambda b, i: (0, 0)),
            pl.BlockSpec((1, b2.shape[0]), lambda b, i: (0, 0)),
        ],
        out_specs=pl.BlockSpec((1, Q_BLK, 3 + W2.shape[1]),
                               lambda b, i: (b, i, 0)),
        out_shape=jax.ShapeDtypeStruct((B, N1, 3 + W2.shape[1]),
                                       jnp.float32),
        compiler_params=pltpu.CompilerParams(
            dimension_semantics=("parallel", "parallel")),
    )(points1, jnp.swapaxes(points2, 1, 2), W0, b0.reshape(1, -1),
      W1, b1.reshape(1, -1), W2, b2.reshape(1, -1))
    return out


# speedup vs baseline: 49.7510x; 1.0318x over previous
import functools

import jax
import jax.numpy as jnp
from jax.experimental import pallas as pl
from jax.experimental.pallas import tpu as pltpu

K_NN = 3
Q_BLK = 4096


def _fused_kernel(p1_ref, p2_ref, w0_ref, b0_ref, w1_ref, b1_ref,
                  w2_ref, b2_ref, out_ref):
    p1 = p1_ref[0]
    p2t = p2_ref[0]
    xyz1 = p1[:, :3]
    feat1 = p1[:, 3:]
    xyz2t = p2t[:3, :]
    feat2t = p2t[3:, :]

    q = xyz1.shape[0]
    n2 = xyz2t.shape[1]

    a2 = jnp.sum(xyz1 * xyz1, axis=1, keepdims=True)
    b2row = jnp.sum(xyz2t * xyz2t, axis=0, keepdims=True)
    cross = jax.lax.dot_general(
        xyz1, xyz2t, (((1,), (0,)), ((), ())),
        preferred_element_type=jnp.float32)
    d2 = a2 + b2row - 2.0 * cross

    inf = jnp.float32(jnp.inf)
    m1 = jnp.min(d2, axis=1, keepdims=True)
    m2 = jnp.min(jnp.where(d2 <= m1, inf, d2), axis=1, keepdims=True)
    m3 = jnp.min(jnp.where(d2 <= m2, inf, d2), axis=1, keepdims=True)

    def _w(v):
        return jax.lax.rsqrt(jnp.maximum(v, 1e-12))
    inv_wsum = 1.0 / (_w(m1) + _w(m2) + _w(m3))
    s = jnp.where(d2 <= m3, _w(d2) * inv_wsum, 0.0)

    interp = jax.lax.dot_general(
        s, feat2t, (((1,), (1,)), ((), ())),
        preferred_element_type=jnp.float32)

    h = jnp.maximum(
        jax.lax.dot_general(feat1, w0_ref[:13, :], (((1,), (0,)), ((), ())),
                            preferred_element_type=jnp.float32)
        + jax.lax.dot_general(interp, w0_ref[13:, :], (((1,), (0,)), ((), ())),
                              preferred_element_type=jnp.float32)
        + b0_ref[:], 0.0)
    h = jnp.maximum(jnp.dot(h, w1_ref[:], preferred_element_type=jnp.float32)
                    + b1_ref[:], 0.0)
    h = jnp.maximum(jnp.dot(h, w2_ref[:], preferred_element_type=jnp.float32)
                    + b2_ref[:], 0.0)
    out_ref[0] = jnp.concatenate([xyz1, h], axis=1)


@jax.jit
def kernel(points1, points2, W0, b0, W1, b1, W2, b2):
    B, N1, C1 = points1.shape
    _, N2, C2 = points2.shape
    grid = (B, N1 // Q_BLK)

    out = pl.pallas_call(
        _fused_kernel,
        grid=grid,
        in_specs=[
            pl.BlockSpec((1, Q_BLK, C1), lambda b, i: (b, i, 0)),
            pl.BlockSpec((1, C2, N2), lambda b, i: (b, 0, 0)),
            pl.BlockSpec(W0.shape, lambda b, i: (0, 0)),
            pl.BlockSpec((1, b0.shape[0]), lambda b, i: (0, 0)),
            pl.BlockSpec(W1.shape, lambda b, i: (0, 0)),
            pl.BlockSpec((1, b1.shape[0]), lambda b, i: (0, 0)),
            pl.BlockSpec(W2.shape, lambda b, i: (0, 0)),
            pl.BlockSpec((1, b2.shape[0]), lambda b, i: (0, 0)),
        ],
        out_specs=pl.BlockSpec((1, Q_BLK, 3 + W2.shape[1]),
                               lambda b, i: (b, i, 0)),
        out_shape=jax.ShapeDtypeStruct((B, N1, 3 + W2.shape[1]),
                                       jnp.float32),
        compiler_params=pltpu.CompilerParams(
            dimension_semantics=("parallel", "parallel")),
    )(points1, jnp.swapaxes(points2, 1, 2), W0, b0.reshape(1, -1),
      W1, b1.reshape(1, -1), W2, b2.reshape(1, -1))
    return out


# speedup vs baseline: 50.3955x; 1.0130x over previous
import functools

import jax
import jax.numpy as jnp
from jax.experimental import pallas as pl
from jax.experimental.pallas import tpu as pltpu

K_NN = 3
Q_BLK = 8192


def _fused_kernel(p1_ref, p2_ref, w0_ref, b0_ref, w1_ref, b1_ref,
                  w2_ref, b2_ref, out_ref):
    p1 = p1_ref[0]
    p2t = p2_ref[0]
    xyz1 = p1[:, :3]
    feat1 = p1[:, 3:]
    xyz2t = p2t[:3, :]
    feat2t = p2t[3:, :]

    q = xyz1.shape[0]
    n2 = xyz2t.shape[1]

    a2 = jnp.sum(xyz1 * xyz1, axis=1, keepdims=True)
    b2row = jnp.sum(xyz2t * xyz2t, axis=0, keepdims=True)
    cross = jax.lax.dot_general(
        xyz1, xyz2t, (((1,), (0,)), ((), ())),
        preferred_element_type=jnp.float32)
    d2 = a2 + b2row - 2.0 * cross

    inf = jnp.float32(jnp.inf)
    m1 = jnp.min(d2, axis=1, keepdims=True)
    m2 = jnp.min(jnp.where(d2 <= m1, inf, d2), axis=1, keepdims=True)
    m3 = jnp.min(jnp.where(d2 <= m2, inf, d2), axis=1, keepdims=True)

    def _w(v):
        return jax.lax.rsqrt(jnp.maximum(v, 1e-12))
    inv_wsum = 1.0 / (_w(m1) + _w(m2) + _w(m3))
    s = jnp.where(d2 <= m3, _w(d2) * inv_wsum, 0.0)

    interp = jax.lax.dot_general(
        s, feat2t, (((1,), (1,)), ((), ())),
        preferred_element_type=jnp.float32)

    h = jnp.maximum(
        jax.lax.dot_general(feat1, w0_ref[:13, :], (((1,), (0,)), ((), ())),
                            preferred_element_type=jnp.float32)
        + jax.lax.dot_general(interp, w0_ref[13:, :], (((1,), (0,)), ((), ())),
                              preferred_element_type=jnp.float32)
        + b0_ref[:], 0.0)
    h = jnp.maximum(jnp.dot(h, w1_ref[:], preferred_element_type=jnp.float32)
                    + b1_ref[:], 0.0)
    h = jnp.maximum(jnp.dot(h, w2_ref[:], preferred_element_type=jnp.float32)
                    + b2_ref[:], 0.0)
    out_ref[0] = jnp.concatenate([xyz1, h], axis=1)


@jax.jit
def kernel(points1, points2, W0, b0, W1, b1, W2, b2):
    B, N1, C1 = points1.shape
    _, N2, C2 = points2.shape
    grid = (B, N1 // Q_BLK)

    out = pl.pallas_call(
        _fused_kernel,
        grid=grid,
        in_specs=[
            pl.BlockSpec((1, Q_BLK, C1), lambda b, i: (b, i, 0)),
            pl.BlockSpec((1, C2, N2), lambda b, i: (b, 0, 0)),
            pl.BlockSpec(W0.shape, lambda b, i: (0, 0)),
            pl.BlockSpec((1, b0.shape[0]), lambda b, i: (0, 0)),
            pl.BlockSpec(W1.shape, lambda b, i: (0, 0)),
            pl.BlockSpec((1, b1.shape[0]), lambda b, i: (0, 0)),
            pl.BlockSpec(W2.shape, lambda b, i: (0, 0)),
            pl.BlockSpec((1, b2.shape[0]), lambda b, i: (0, 0)),
        ],
        out_specs=pl.BlockSpec((1, Q_BLK, 3 + W2.shape[1]),
                               lambda b, i: (b, i, 0)),
        out_shape=jax.ShapeDtypeStruct((B, N1, 3 + W2.shape[1]),
                                       jnp.float32),
        compiler_params=pltpu.CompilerParams(
            dimension_semantics=("parallel", "parallel")),
    )(points1, jnp.swapaxes(points2, 1, 2), W0, b0.reshape(1, -1),
      W1, b1.reshape(1, -1), W2, b2.reshape(1, -1))
    return out


# speedup vs baseline: 51.5816x; 1.0235x over previous
import functools

import jax
import jax.numpy as jnp
from jax.experimental import pallas as pl
from jax.experimental.pallas import tpu as pltpu

K_NN = 3
Q_BLK = 8192


def _fused_kernel(p1_ref, p2_ref, w0_ref, b0_ref, w1_ref, b1_ref,
                  w2_ref, b2_ref, out_ref):
    p1 = p1_ref[0]
    p2t = p2_ref[0]
    xyz1 = p1[:, :3]
    feat1 = p1[:, 3:]
    xyz2t = p2t[:3, :]
    feat2t = p2t[3:, :]

    q = xyz1.shape[0]
    n2 = xyz2t.shape[1]

    a2 = jnp.sum(xyz1 * xyz1, axis=1, keepdims=True)
    b2row = jnp.sum(xyz2t * xyz2t, axis=0, keepdims=True)
    cross = jax.lax.dot_general(
        xyz1, xyz2t, (((1,), (0,)), ((), ())),
        preferred_element_type=jnp.float32)
    d2 = a2 + b2row - 2.0 * cross

    inf = jnp.float32(jnp.inf)
    m1 = jnp.min(d2, axis=1, keepdims=True)
    m2 = jnp.min(jnp.where(d2 <= m1, inf, d2), axis=1, keepdims=True)
    m3 = jnp.min(jnp.where(d2 <= m2, inf, d2), axis=1, keepdims=True)

    def _w(v):
        return jax.lax.rsqrt(jnp.maximum(v, 1e-12))
    inv_wsum = 1.0 / (_w(m1) + _w(m2) + _w(m3))
    s = jnp.where(d2 <= m3, _w(d2), 0.0)

    interp = jax.lax.dot_general(
        s.astype(jnp.bfloat16), feat2t.astype(jnp.bfloat16),
        (((1,), (1,)), ((), ())),
        preferred_element_type=jnp.float32) * inv_wsum

    h = jnp.maximum(
        jax.lax.dot_general(feat1, w0_ref[:13, :], (((1,), (0,)), ((), ())),
                            preferred_element_type=jnp.float32)
        + jax.lax.dot_general(interp, w0_ref[13:, :], (((1,), (0,)), ((), ())),
                              preferred_element_type=jnp.float32)
        + b0_ref[:], 0.0)
    h = jnp.maximum(jnp.dot(h, w1_ref[:], preferred_element_type=jnp.float32)
                    + b1_ref[:], 0.0)
    h = jnp.maximum(jnp.dot(h, w2_ref[:], preferred_element_type=jnp.float32)
                    + b2_ref[:], 0.0)
    out_ref[0] = jnp.concatenate([xyz1, h], axis=1)


@jax.jit
def kernel(points1, points2, W0, b0, W1, b1, W2, b2):
    B, N1, C1 = points1.shape
    _, N2, C2 = points2.shape
    grid = (B, N1 // Q_BLK)

    out = pl.pallas_call(
        _fused_kernel,
        grid=grid,
        in_specs=[
            pl.BlockSpec((1, Q_BLK, C1), lambda b, i: (b, i, 0)),
            pl.BlockSpec((1, C2, N2), lambda b, i: (b, 0, 0)),
            pl.BlockSpec(W0.shape, lambda b, i: (0, 0)),
            pl.BlockSpec((1, b0.shape[0]), lambda b, i: (0, 0)),
            pl.BlockSpec(W1.shape, lambda b, i: (0, 0)),
            pl.BlockSpec((1, b1.shape[0]), lambda b, i: (0, 0)),
            pl.BlockSpec(W2.shape, lambda b, i: (0, 0)),
            pl.BlockSpec((1, b2.shape[0]), lambda b, i: (0, 0)),
        ],
        out_specs=pl.BlockSpec((1, Q_BLK, 3 + W2.shape[1]),
                               lambda b, i: (b, i, 0)),
        out_shape=jax.ShapeDtypeStruct((B, N1, 3 + W2.shape[1]),
                                       jnp.float32),
        compiler_params=pltpu.CompilerParams(
            dimension_semantics=("parallel", "parallel")),
    )(points1, jnp.swapaxes(points2, 1, 2), W0, b0.reshape(1, -1),
      W1, b1.reshape(1, -1), W2, b2.reshape(1, -1))
    return out
